# baseline (speedup 1.0000x reference)
import functools

import jax
import jax.numpy as jnp
import numpy as np
from jax.experimental import pallas as pl
from jax.experimental.pallas import tpu as pltpu

_N_BINS = 15
_NB_BOUNDS = _N_BINS + 1
_BOUNDS = [float(x) for x in np.linspace(0.0, 1.0, _NB_BOUNDS, dtype=np.float32)]


def _ece_body(yp_ref, yt_ref, out_ref, acc_ref, *, inv_n: float):
    i = pl.program_id(0)
    nb = pl.num_programs(0)

    @pl.when(i == 0)
    def _init():
        acc_ref[...] = jnp.zeros_like(acc_ref)

    logits = yp_ref[...]
    e = jnp.exp(logits)
    se = jnp.sum(e, axis=1)
    m = jnp.max(logits, axis=1)
    conf = jnp.exp(m) / se
    pred = jnp.argmax(logits, axis=1)
    correct = pred == yt_ref[0, 0, :]

    for k in range(_NB_BOUNDS):
        g = conf > _BOUNDS[k]
        acc_ref[k, :] += g.astype(jnp.float32)
        acc_ref[_NB_BOUNDS + k, :] += jnp.where(g, conf, 0.0)
        acc_ref[2 * _NB_BOUNDS + k, :] += jnp.where(g & correct, 1.0, 0.0)

    @pl.when(i == nb - 1)
    def _fin():
        r = jnp.sum(acc_ref[...], axis=1)
        cnt = r[0:_N_BINS] - r[1:_N_BINS + 1]
        dcf = r[_NB_BOUNDS:_NB_BOUNDS + _N_BINS] - r[_NB_BOUNDS + 1:_NB_BOUNDS + _N_BINS + 1]
        dac = r[2 * _NB_BOUNDS:2 * _NB_BOUNDS + _N_BINS] - r[2 * _NB_BOUNDS + 1:2 * _NB_BOUNDS + _N_BINS + 1]
        denom = jnp.maximum(cnt, 1.0)
        term = jnp.where(cnt > 0, jnp.abs(dcf - dac) / denom * (cnt * inv_n), 0.0)
        out_ref[...] = jnp.broadcast_to(jnp.sum(term)[None, None], (1, 128))


def kernel(y_pred, y_true):
    n, c = y_pred.shape
    bn = 8000
    nb = n // bn
    yt = y_true.astype(jnp.int32).reshape(nb, 1, bn)

    out = pl.pallas_call(
        functools.partial(_ece_body, inv_n=1.0 / n),
        grid=(nb,),
        in_specs=[
            pl.BlockSpec((bn, c), lambda i: (i, 0)),
            pl.BlockSpec((1, 1, bn), lambda i: (i, 0, 0)),
        ],
        out_specs=pl.BlockSpec((1, 128), lambda i: (0, 0)),
        out_shape=jax.ShapeDtypeStruct((1, 128), jnp.float32),
        scratch_shapes=[pltpu.VMEM((3 * _NB_BOUNDS, bn), jnp.float32)],
        compiler_params=pltpu.CompilerParams(
            dimension_semantics=("arbitrary",),
        ),
    )(y_pred, yt)
    return out[0, 0:1]


# speedup vs baseline: 16.6385x; 16.6385x over previous
import functools

import jax
import jax.numpy as jnp
from jax.experimental import pallas as pl
from jax.experimental.pallas import tpu as pltpu

_N_BINS = 15


def _pick_bn(n: int) -> int:
    best = 8
    for bn in range(8, 8193, 8):
        if n % bn == 0:
            best = bn
    return best


def _ece_body(bounds_ref, yp_ref, yt_ref, out_ref, acc_ref, *, inv_n: float):
    i = pl.program_id(0)
    nb = pl.num_programs(0)

    @pl.when(i == 0)
    def _init():
        acc_ref[...] = jnp.zeros_like(acc_ref)

    logits = yp_ref[...]
    m = jnp.max(logits, axis=1, keepdims=True)
    se = jnp.sum(jnp.exp(logits - m), axis=1, keepdims=True)
    conf = 1.0 / se
    pred = jnp.argmax(logits, axis=1)[:, None]
    correct = (pred == yt_ref[...]).astype(jnp.float32)

    g = (conf > bounds_ref[...]).astype(jnp.float32)
    s_cnt = jnp.sum(g, axis=0, keepdims=True)
    s_conf = jnp.sum(g * conf, axis=0, keepdims=True)
    s_acc = jnp.sum(g * correct, axis=0, keepdims=True)
    acc_ref[0:1, :] += s_cnt
    acc_ref[1:2, :] += s_conf
    acc_ref[2:3, :] += s_acc

    @pl.when(i == nb - 1)
    def _fin():
        cnt = acc_ref[0:1, 0:_N_BINS] - acc_ref[0:1, 1:_N_BINS + 1]
        dcf = acc_ref[1:2, 0:_N_BINS] - acc_ref[1:2, 1:_N_BINS + 1]
        dac = acc_ref[2:3, 0:_N_BINS] - acc_ref[2:3, 1:_N_BINS + 1]
        denom = jnp.maximum(cnt, 1.0)
        term = jnp.where(cnt > 0, jnp.abs(dcf - dac) / denom * (cnt * inv_n), 0.0)
        ece = jnp.sum(term, axis=1, keepdims=True)
        out_ref[...] = jnp.broadcast_to(ece, (1, 128))


def kernel(y_pred, y_true):
    n, c = y_pred.shape
    bn = _pick_bn(n)
    nb = n // bn

    bb = jnp.linspace(0.0, 1.0, _N_BINS + 1).astype(jnp.float32)
    bounds = jnp.concatenate(
        [bb, jnp.full((128 - (_N_BINS + 1),), 2.0, jnp.float32)]
    ).reshape(1, 128)
    yt = y_true.astype(jnp.int32).reshape(n, 1)

    out = pl.pallas_call(
        functools.partial(_ece_body, inv_n=1.0 / n),
        grid=(nb,),
        in_specs=[
            pl.BlockSpec((1, 128), lambda i: (0, 0)),
            pl.BlockSpec((bn, c), lambda i: (i, 0)),
            pl.BlockSpec((bn, 1), lambda i: (i, 0)),
        ],
        out_specs=pl.BlockSpec((1, 128), lambda i: (0, 0)),
        out_shape=jax.ShapeDtypeStruct((1, 128), jnp.float32),
        scratch_shapes=[pltpu.VMEM((8, 128), jnp.float32)],
        compiler_params=pltpu.CompilerParams(
            dimension_semantics=("arbitrary",),
        ),
    )(bounds, y_pred, yt)
    return out[0, 0:1]


# speedup vs baseline: 20.9899x; 1.2615x over previous
import functools

import jax
import jax.numpy as jnp
from jax.experimental import pallas as pl
from jax.experimental.pallas import tpu as pltpu

_N_BINS = 15
_BN = 8192


def _accum(conf, correct, bounds, acc_ref):
    g = (conf > bounds).astype(jnp.float32)
    acc_ref[0:1, :] += jnp.sum(g, axis=0, keepdims=True)
    acc_ref[1:2, :] += jnp.sum(g * conf, axis=0, keepdims=True)
    acc_ref[2:3, :] += jnp.sum(g * correct, axis=0, keepdims=True)


def _ece_body(bounds_ref, yp_ref, yt_ref, out_ref, acc_ref, *, n_rows: int):
    i = pl.program_id(0)
    nb = pl.num_programs(0)

    @pl.when(i == 0)
    def _init():
        acc_ref[...] = jnp.zeros_like(acc_ref)

    logits = yp_ref[...]
    m = jnp.max(logits, axis=1, keepdims=True)
    se = jnp.sum(jnp.exp(logits - m), axis=1, keepdims=True)
    conf = 1.0 / se
    pred = jnp.argmax(logits, axis=1)[:, None]
    yt = yt_ref[...][:, None]
    correct = (pred == yt).astype(jnp.float32)
    bounds = bounds_ref[...]

    @pl.when(i < nb - 1)
    def _acc_full():
        _accum(conf, correct, bounds, acc_ref)

    @pl.when(i == nb - 1)
    def _acc_tail():
        valid = n_rows - (nb - 1) * _BN
        rloc = jax.lax.broadcasted_iota(jnp.int32, (_BN, 1), 0)
        _accum(jnp.where(rloc < valid, conf, 2.0), correct, bounds, acc_ref)

    @pl.when(i == nb - 1)
    def _fin():
        cnt = acc_ref[0:1, 0:_N_BINS] - acc_ref[0:1, 1:_N_BINS + 1]
        dcf = acc_ref[1:2, 0:_N_BINS] - acc_ref[1:2, 1:_N_BINS + 1]
        dac = acc_ref[2:3, 0:_N_BINS] - acc_ref[2:3, 1:_N_BINS + 1]
        denom = jnp.maximum(cnt, 1.0)
        term = jnp.where(cnt > 0, jnp.abs(dcf - dac) / denom * (cnt * (1.0 / n_rows)), 0.0)
        ece = jnp.sum(term, axis=1, keepdims=True)
        out_ref[...] = jnp.broadcast_to(ece, (1, 128))


def kernel(y_pred, y_true):
    n, c = y_pred.shape
    nb = (n + _BN - 1) // _BN

    bb = jnp.linspace(0.0, 1.0, _N_BINS + 1).astype(jnp.float32)
    bounds = jnp.concatenate(
        [bb, jnp.full((128 - (_N_BINS + 1),), 2.0, jnp.float32)]
    ).reshape(1, 128)
    yt = y_true.astype(jnp.int32)

    out = pl.pallas_call(
        functools.partial(_ece_body, n_rows=n),
        grid=(nb,),
        in_specs=[
            pl.BlockSpec((1, 128), lambda i: (0, 0)),
            pl.BlockSpec((_BN, c), lambda i: (i, 0)),
            pl.BlockSpec((_BN,), lambda i: (i,)),
        ],
        out_specs=pl.BlockSpec((1, 128), lambda i: (0, 0)),
        out_shape=jax.ShapeDtypeStruct((1, 128), jnp.float32),
        scratch_shapes=[pltpu.VMEM((8, 128), jnp.float32)],
        compiler_params=pltpu.CompilerParams(
            dimension_semantics=("arbitrary",),
        ),
    )(bounds, y_pred, yt)
    return out[0, 0:1]
